# Initial kernel scaffold; baseline (speedup 1.0000x reference)
import jax
import jax.numpy as jnp
from jax.experimental import pallas as pl


def kernel(x_p1, edge_index_p1, edge_attr_p1, u_p1, batch_p1, x_p2, edge_index_p2, edge_attr_p2, u_p2, batch_p2, x_pm, edge_index_pm, edge_attr_pm, u_pm, batch_pm, Temperature, params):
    raise NotImplementedError("write your pallas kernel here")


# speedup vs baseline: 7.5245x; 7.5245x over previous
import functools

import jax
import jax.numpy as jnp
from jax import lax
from jax.experimental import pallas as pl
from jax.experimental.pallas import tpu as pltpu
from jax.experimental.pallas import tpu_sc as plsc

N = 50000
E = 800000
B = 64
NC = 2
NS = 16
NW = NC * NS
CH = 128
NCHUNK = E // CH
R0 = 3128
LAST_OFF = (NS - 1) * R0
LAST_N = N - LAST_OFF
BLK = 2000
NBLK = N // BLK

def _striped(s, fn):
    @pl.when(s < NS - 1)
    def _():
        fn(pl.multiple_of(s * R0, 8), R0)

    @pl.when(s == NS - 1)
    def _():
        fn(LAST_OFF, LAST_N)


@functools.cache
def _sc_kernels():
    mesh = plsc.VectorSubcoreMesh(core_axis_name="c", subcore_axis_name="s")

    @functools.partial(
        pl.kernel,
        out_type=jax.ShapeDtypeStruct((NC, N, 32), jnp.float32),
        mesh=mesh,
        compiler_params=pltpu.CompilerParams(use_tc_tiling_on_sc=False),
        scratch_types=[
            pltpu.VMEM((CH,), jnp.int32),
            pltpu.VMEM((CH,), jnp.int32),
            pltpu.VMEM((CH, 32), jnp.float32),
            pltpu.VMEM_SHARED((N, 32), jnp.float32),
            pltpu.SemaphoreType.DMA,
        ],
    )
    def _gather_add(xcat, rowi, coli, zer, out, rowb, colb, xrows, acc, sem):
        c = lax.axis_index("c")
        s = lax.axis_index("s")
        _striped(s, lambda o, n: pltpu.sync_copy(zer.at[pl.ds(o, n), :],
                                                 acc.at[pl.ds(o, n), :]))
        plsc.subcore_barrier()
        base = c * N
        nch = (NCHUNK + NS - 1) // NS

        def body(i, carry):
            j = s + i * NS

            @pl.when(j < NCHUNK)
            def _():
                off = j * CH
                pltpu.sync_copy(rowi.at[pl.ds(off, CH)], rowb)
                pltpu.sync_copy(coli.at[pl.ds(off, CH)], colb)
                for k in range(CH // 16):
                    sl = pl.ds(k * 16, 16)
                    rowb[sl] = rowb[sl] + base
                pltpu.async_copy(xcat.at[rowb], xrows, sem).wait()
                pltpu.sync_copy(xrows, acc.at[colb], add=True)

            return carry

        lax.fori_loop(0, nch, body, 0)
        plsc.subcore_barrier()
        _striped(s, lambda o, n: pltpu.sync_copy(acc.at[pl.ds(o, n), :],
                                                 out.at[c, pl.ds(o, n), :]))

    @functools.partial(
        pl.kernel,
        out_type=jax.ShapeDtypeStruct((NC, 3, N, 8), jnp.float32),
        mesh=mesh,
        compiler_params=pltpu.CompilerParams(use_tc_tiling_on_sc=False),
        scratch_types=[
            pltpu.VMEM((CH,), jnp.int32),
            pltpu.VMEM((CH, 8), jnp.float32),
            pltpu.VMEM_SHARED((N, 8), jnp.float32),
            pltpu.VMEM_SHARED((N, 8), jnp.float32),
            pltpu.VMEM_SHARED((N, 8), jnp.float32),
            pltpu.SemaphoreType.DMA,
        ],
    )
    def _edge_seg(ea0, col0, ea1, col1, ea2, col2, zer, out,
                  colb, erows, a0, a1, a2, sem):
        c = lax.axis_index("c")
        s = lax.axis_index("s")
        w = s * NC + c
        for acc in (a0, a1, a2):
            _striped(s, lambda o, n, acc=acc: pltpu.sync_copy(
                zer.at[pl.ds(o, n), :], acc.at[pl.ds(o, n), :]))
        plsc.subcore_barrier()
        nch = (NCHUNK + NW - 1) // NW
        for ea, coli, acc in ((ea0, col0, a0), (ea1, col1, a1), (ea2, col2, a2)):

            def body(i, carry, ea=ea, coli=coli, acc=acc):
                j = w + i * NW

                @pl.when(j < NCHUNK)
                def _():
                    off = j * CH
                    pltpu.sync_copy(coli.at[pl.ds(off, CH)], colb)
                    pltpu.sync_copy(ea.at[pl.ds(off, CH), :], erows)
                    pltpu.sync_copy(erows, acc.at[colb], add=True)

                return carry

            lax.fori_loop(0, nch, body, 0)
        plsc.subcore_barrier()
        for g, acc in enumerate((a0, a1, a2)):
            _striped(s, lambda o, n, g=g, acc=acc: pltpu.sync_copy(
                acc.at[pl.ds(o, n), :], out.at[c, g, pl.ds(o, n), :]))

    return _gather_add, _edge_seg


def _dot(a, b):
    return jnp.dot(a, b, preferred_element_type=jnp.float32)


def _enc_body(x_ref, w_ref, b_ref, o_ref):
    y = _dot(x_ref[...], w_ref[...]) + b_ref[...]
    o_ref[0] = y[:, :32]
    o_ref[1] = y[:, 32:]


def _encode(xraw, w, b):
    return pl.pallas_call(
        _enc_body,
        grid=(NBLK,),
        in_specs=[pl.BlockSpec((BLK, 17), lambda i: (i, 0)),
                  pl.BlockSpec((17, 64), lambda i: (0, 0)),
                  pl.BlockSpec((1, 64), lambda i: (0, 0))],
        out_specs=pl.BlockSpec((2, BLK, 32), lambda i: (0, i, 0)),
        out_shape=jax.ShapeDtypeStruct((2, N, 32), jnp.float32),
    )(xraw, w, b)


def _k1_body(xst, ax, sacc, wer, wec, m7, v7, w1x, w1e, b1, w2x, w2a, b2,
             x1st, s1o):
    x0 = jnp.concatenate([xst[0], xst[1]], axis=1)
    ax0 = jnp.concatenate([ax[0], ax[1]], axis=1)
    sc = sacc[0, 0] + sacc[1, 0]
    deg = sc[:, 7:8]
    s1 = (_dot(ax0, wer[...]) + _dot(deg * x0, wec[...])
          + _dot(sc[:, 0:7], m7[...]) + deg * v7[...])
    agg1 = _dot(ax0, w1x[...]) + _dot(s1, w1e[...]) + deg * b1[...]
    x1 = _dot(x0, w2x[...]) + _dot(agg1, w2a[...]) + b2[...]
    x1st[0] = x1[:, :32]
    x1st[1] = x1[:, 32:]
    s1o[...] = s1


def _round1(g, xst, ax0, sacc, ws):
    wspec = [pl.BlockSpec(w.shape, lambda i: (0,) * w.ndim) for w in ws]
    return pl.pallas_call(
        _k1_body,
        grid=(NBLK,),
        in_specs=[pl.BlockSpec((2, BLK, 32), lambda i: (0, i, 0)),
                  pl.BlockSpec((2, BLK, 32), lambda i: (0, i, 0)),
                  pl.BlockSpec((2, 1, BLK, 8), lambda i, g=g: (0, g, i, 0))]
        + wspec,
        out_specs=[pl.BlockSpec((2, BLK, 32), lambda i: (0, i, 0)),
                   pl.BlockSpec((BLK, 32), lambda i: (i, 0))],
        out_shape=[jax.ShapeDtypeStruct((2, N, 32), jnp.float32),
                   jax.ShapeDtypeStruct((N, 32), jnp.float32)],
    )(xst, ax0, sacc, *ws)


def _k2_body(x1st, ax, s1, sacc, bt, wer, wec, wea, bev, w1x, w1e, b1,
             w2x, w2a, b2, nacc, eacc):
    i = pl.program_id(0)
    x1 = jnp.concatenate([x1st[0], x1st[1]], axis=1)
    ax1 = jnp.concatenate([ax[0], ax[1]], axis=1)
    sc = sacc[0, 0] + sacc[1, 0]
    deg = sc[:, 7:8]
    s2 = (_dot(ax1, wer[...]) + _dot(deg * x1, wec[...])
          + _dot(s1[...], wea[...]) + deg * bev[...])
    agg2 = _dot(ax1, w1x[...]) + _dot(s2, w1e[...]) + deg * b1[...]
    x2 = _dot(x1, w2x[...]) + _dot(agg2, w2a[...]) + b2[...]
    bvec = bt[0]
    oh = (lax.broadcasted_iota(jnp.int32, (B, 1), 0) == bvec).astype(jnp.float32)
    part_n = _dot(oh, x2)
    part_e = _dot(oh, s2)

    @pl.when(i == 0)
    def _():
        nacc[...] = jnp.zeros_like(nacc)
        eacc[...] = jnp.zeros_like(eacc)

    nacc[...] += part_n
    eacc[...] += part_e


def _round2(g, x1st, ax1, s1, sacc, bt3, ws):
    wspec = [pl.BlockSpec(w.shape, lambda i: (0,) * w.ndim) for w in ws]
    return pl.pallas_call(
        _k2_body,
        grid=(NBLK,),
        in_specs=[pl.BlockSpec((2, BLK, 32), lambda i: (0, i, 0)),
                  pl.BlockSpec((2, BLK, 32), lambda i: (0, i, 0)),
                  pl.BlockSpec((BLK, 32), lambda i: (i, 0)),
                  pl.BlockSpec((2, 1, BLK, 8), lambda i, g=g: (0, g, i, 0)),
                  pl.BlockSpec((1, 1, BLK), lambda i: (i, 0, 0))]
        + wspec,
        out_specs=[pl.BlockSpec((B, 64), lambda i: (0, 0)),
                   pl.BlockSpec((B, 32), lambda i: (0, 0))],
        out_shape=[jax.ShapeDtypeStruct((B, 64), jnp.float32),
                   jax.ShapeDtypeStruct((B, 32), jnp.float32)],
    )(x1st, ax1, s1, sacc, bt3, *ws)


def _fin_body(na0, ea0, na1, ea1, na2, ea2, up1, up2, upm, tt,
              wgn, wge, bg, wn2, bn2, we2t, we2d, be2,
              w3er, w3ec, w3ea, b3e, w31x, w31e, b31, w32x, w32a, b32,
              w3gn, w3ge, b3g, wl1, bl1, lng, lnb, wl2, bl2, out):
    xs = []
    for na, ea in ((na0, ea0), (na1, ea1), (na2, ea2)):
        u = _dot(na[...], wgn[...]) + _dot(ea[...], wge[...]) + bg[...]
        xs.append(_dot(u, wn2[...]) + bn2[...])
    x0, x1, x2 = xs
    p1d = up1[...] / upm[...]
    p2d = up2[...] / upm[...]
    t = tt[...]
    c1 = _dot(t, we2t[...]) + _dot(p1d, we2d[...]) + be2[...]
    c2 = _dot(t, we2t[...]) + _dot(p2d, we2d[...]) + be2[...]
    e0, e1, e2, e3 = c1, c1, c2, c2
    for _ in range(2):
        n0 = _dot(x0, w3er[...]) + _dot(x2, w3ec[...]) + _dot(e0, w3ea[...]) + b3e[...]
        n1 = _dot(x2, w3er[...]) + _dot(x0, w3ec[...]) + _dot(e1, w3ea[...]) + b3e[...]
        n2 = _dot(x1, w3er[...]) + _dot(x2, w3ec[...]) + _dot(e2, w3ea[...]) + b3e[...]
        n3 = _dot(x2, w3er[...]) + _dot(x1, w3ec[...]) + _dot(e3, w3ea[...]) + b3e[...]
        h0 = _dot(x0, w31x[...]) + _dot(n0, w31e[...]) + b31[...]
        h1 = _dot(x2, w31x[...]) + _dot(n1, w31e[...]) + b31[...]
        h2 = _dot(x1, w31x[...]) + _dot(n2, w31e[...]) + b31[...]
        h3 = _dot(x2, w31x[...]) + _dot(n3, w31e[...]) + b31[...]
        x0 = _dot(x0, w32x[...]) + _dot(h1, w32a[...]) + b32[...]
        x1 = _dot(x1, w32x[...]) + _dot(h3, w32a[...]) + b32[...]
        x2 = _dot(x2, w32x[...]) + _dot(h0 + h2, w32a[...]) + b32[...]
        e0, e1, e2, e3 = n0, n1, n2, n3
    gamma = (_dot(x0 + x1 + x2, w3gn[...])
             + _dot(e0 + e1 + e2 + e3, w3ge[...]) + b3g[...])
    h = _dot(gamma, wl1[...]) + bl1[...]
    h = jnp.where(h > 0, h, 0.01 * h)
    mu = jnp.mean(h, axis=-1, keepdims=True)
    d = h - mu
    var = jnp.mean(d * d, axis=-1, keepdims=True)
    h = d / jnp.sqrt(var + 1e-5) * lng[...] + lnb[...]
    out[...] = _dot(h, wl2[...]) + bl2[...]


def _final(args):
    specs = [pl.BlockSpec(a.shape, lambda i: (0,) * a.ndim) for a in args]
    return pl.pallas_call(
        _fin_body,
        grid=(1,),
        in_specs=specs,
        out_specs=pl.BlockSpec((B, 15), lambda i: (0, 0)),
        out_shape=jax.ShapeDtypeStruct((B, 15), jnp.float32),
    )(*args)


def kernel(x_p1, edge_index_p1, edge_attr_p1, u_p1, batch_p1,
           x_p2, edge_index_p2, edge_attr_p2, u_p2, batch_p2,
           x_pm, edge_index_pm, edge_attr_pm, u_pm, batch_pm,
           Temperature, params):
    p = params
    f32 = jnp.float32

    wer = p["m1_edge"]["W"][0:64]
    wec = p["m1_edge"]["W"][64:128]
    wea = p["m1_edge"]["W"][128:160]
    be = p["m1_edge"]["b"]
    w1x = p["m1_n1"]["W"][0:64]
    w1e = p["m1_n1"]["W"][64:96]
    b1 = p["m1_n1"]["b"][None, :]
    w2x = p["m1_n2"]["W"][0:64]
    w2a = p["m1_n2"]["W"][64:128]
    b2 = p["m1_n2"]["b"][None, :]
    m7 = p["enc_e1"]["W"] @ wea
    v7 = (p["enc_e1"]["b"] @ wea + be)[None, :]
    ws1 = [wer, wec, m7, v7, w1x, w1e, b1, w2x, w2a, b2]
    ws2 = [wer, wec, wea, be[None, :], w1x, w1e, b1, w2x, w2a, b2]

    graphs = [(x_p1, edge_index_p1, edge_attr_p1, batch_p1),
              (x_p2, edge_index_p2, edge_attr_p2, batch_p2),
              (x_pm, edge_index_pm, edge_attr_pm, batch_pm)]
    ones_col = jnp.ones((E, 1), f32)
    ea8s = [jnp.concatenate([g[2], ones_col], axis=1) for g in graphs]
    cols = [g[1][1] for g in graphs]
    rows = [g[1][0] for g in graphs]
    zer8 = jnp.zeros((N, 8), f32)
    zer32 = jnp.zeros((N, 32), f32)

    sc_gather_add, sc_edge_seg = _sc_kernels()
    sacc = sc_edge_seg(ea8s[0], cols[0], ea8s[1], cols[1], ea8s[2], cols[2],
                       zer8)

    wenc = p["enc_n1"]["W"]
    benc = p["enc_n1"]["b"][None, :]
    fin_args = []
    for g in range(3):
        xraw, _, _, bt = graphs[g]
        xst0 = _encode(xraw, wenc, benc)
        ax0 = sc_gather_add(xst0.reshape(2 * N, 32), rows[g], cols[g], zer32)
        x1st, s1 = _round1(g, xst0, ax0, sacc, ws1)
        ax1 = sc_gather_add(x1st.reshape(2 * N, 32), rows[g], cols[g], zer32)
        bt3 = bt.reshape(NBLK, 1, BLK)
        nacc, eacc = _round2(g, x1st, ax1, s1, sacc, bt3, ws2)
        fin_args += [nacc, eacc]

    fin_args += [u_p1[:, None], u_p2[:, None], u_pm[:, None],
                 Temperature[:, None],
                 p["m1_glob"]["W"][0:64], p["m1_glob"]["W"][64:96],
                 p["m1_glob"]["b"][None, :],
                 p["enc_n2"]["W"], p["enc_n2"]["b"][None, :],
                 p["enc_e2"]["W"][0:1], p["enc_e2"]["W"][1:2],
                 p["enc_e2"]["b"][None, :],
                 p["m3_edge"]["W"][0:128], p["m3_edge"]["W"][128:256],
                 p["m3_edge"]["W"][256:288], p["m3_edge"]["b"][None, :],
                 p["m3_n1"]["W"][0:128], p["m3_n1"]["W"][128:160],
                 p["m3_n1"]["b"][None, :],
                 p["m3_n2"]["W"][0:128], p["m3_n2"]["W"][128:256],
                 p["m3_n2"]["b"][None, :],
                 p["m3_glob"]["W"][0:128], p["m3_glob"]["W"][128:160],
                 p["m3_glob"]["b"][None, :],
                 p["last1"]["W"], p["last1"]["b"][None, :],
                 p["ln"]["g"][None, :], p["ln"]["b"][None, :],
                 p["last2"]["W"], p["last2"]["b"][None, :]]
    return _final(fin_args)


# speedup vs baseline: 13.3117x; 1.7691x over previous
import functools

import jax
import jax.numpy as jnp
from jax import lax
from jax.experimental import pallas as pl
from jax.experimental.pallas import tpu as pltpu
from jax.experimental.pallas import tpu_sc as plsc

N = 50000
E = 800000
B = 64
NC = 2
NS = 16
NW = NC * NS
CH = 128
NCHUNK = E // CH
R0 = 3128
LAST_OFF = (NS - 1) * R0
LAST_N = N - LAST_OFF
CPT = -(-NCHUNK // NS)
CPW = -(-NCHUNK // NW)
IB = 17
NBT = CPT // IB
LAST_IB = NCHUNK - ((NS - 1) * CPT + (NBT - 1) * IB)
BLK = 2000
NBLK = N // BLK

def _striped(s, fn):
    @pl.when(s < NS - 1)
    def _():
        fn(pl.multiple_of(s * R0, 8), R0)

    @pl.when(s == NS - 1)
    def _():
        fn(LAST_OFF, LAST_N)


@functools.cache
def _sc_kernels():
    mesh = plsc.VectorSubcoreMesh(core_axis_name="c", subcore_axis_name="s")

    @functools.partial(
        pl.kernel,
        out_type=jax.ShapeDtypeStruct((NC, N, 32), jnp.float32),
        mesh=mesh,
        compiler_params=pltpu.CompilerParams(use_tc_tiling_on_sc=False),
        scratch_types=[
            pltpu.VMEM((IB, CH), jnp.int32),
            pltpu.VMEM((IB, CH), jnp.int32),
            pltpu.VMEM((CH, 32), jnp.float32),
            pltpu.VMEM((CH, 32), jnp.float32),
            pltpu.VMEM_SHARED((N, 32), jnp.float32),
            pltpu.SemaphoreType.DMA,
            pltpu.SemaphoreType.DMA,
        ],
    )
    def _gather_add(xst, rowi, coli, zer, out,
                    rowb, colb, buf0, buf1, acc, sem0, sem1):
        c = lax.axis_index("c")
        s = lax.axis_index("s")
        _striped(s, lambda o, n: pltpu.sync_copy(zer.at[pl.ds(o, n), :],
                                                 acc.at[pl.ds(o, n), :]))
        plsc.subcore_barrier()
        j0 = s * CPT
        cnt = jnp.minimum(CPT, NCHUNK - j0)
        tbl = xst.at[c]

        def start(i, buf, sem):
            pltpu.async_copy(tbl.at[rowb.at[i]], buf, sem)

        def finish(i, buf, sem):
            pltpu.make_async_copy(tbl.at[rowb.at[i]], buf, sem).wait()
            pltpu.sync_copy(buf, acc.at[colb.at[i]], add=True)

        def blk(b, carry):
            ib0 = b * IB
            o = j0 + ib0
            tail = (s == NS - 1) & (b == NBT - 1)

            @pl.when(jnp.logical_not(tail))
            def _():
                pltpu.sync_copy(rowi.at[pl.ds(o, IB), :], rowb)
                pltpu.sync_copy(coli.at[pl.ds(o, IB), :], colb)

            @pl.when(tail)
            def _():
                pltpu.sync_copy(rowi.at[pl.ds(o, LAST_IB), :],
                                rowb.at[pl.ds(0, LAST_IB), :])
                pltpu.sync_copy(coli.at[pl.ds(o, LAST_IB), :],
                                colb.at[pl.ds(0, LAST_IB), :])

            @pl.when(ib0 < cnt)
            def _():
                start(0, buf0, sem0)

            for kp in range((IB + 1) // 2):
                k0, k1, k2 = 2 * kp, 2 * kp + 1, 2 * kp + 2
                if k1 < IB:
                    @pl.when(ib0 + k1 < cnt)
                    def _(k1=k1):
                        start(k1, buf1, sem1)

                @pl.when(ib0 + k0 < cnt)
                def _(k0=k0):
                    finish(k0, buf0, sem0)

                if k2 < IB:
                    @pl.when(ib0 + k2 < cnt)
                    def _(k2=k2):
                        start(k2, buf0, sem0)

                if k1 < IB:
                    @pl.when(ib0 + k1 < cnt)
                    def _(k1=k1):
                        finish(k1, buf1, sem1)

            return carry

        lax.fori_loop(0, NBT, blk, 0)
        plsc.subcore_barrier()
        _striped(s, lambda o, n: pltpu.sync_copy(acc.at[pl.ds(o, n), :],
                                                 out.at[c, pl.ds(o, n), :]))

    @functools.partial(
        pl.kernel,
        out_type=jax.ShapeDtypeStruct((NC, 3, N, 8), jnp.float32),
        mesh=mesh,
        compiler_params=pltpu.CompilerParams(use_tc_tiling_on_sc=False),
        scratch_types=[
            pltpu.VMEM((CPW, CH), jnp.int32),
            pltpu.VMEM((CH, 8), jnp.float32),
            pltpu.VMEM((CH, 8), jnp.float32),
            pltpu.VMEM_SHARED((N, 8), jnp.float32),
            pltpu.VMEM_SHARED((N, 8), jnp.float32),
            pltpu.VMEM_SHARED((N, 8), jnp.float32),
            pltpu.SemaphoreType.DMA,
            pltpu.SemaphoreType.DMA,
        ],
    )
    def _edge_seg(ea0, col0, ea1, col1, ea2, col2, zer, out,
                  colb, ebuf0, ebuf1, a0, a1, a2, sem0, sem1):
        c = lax.axis_index("c")
        s = lax.axis_index("s")
        w = s * NC + c
        for acc in (a0, a1, a2):
            _striped(s, lambda o, n, acc=acc: pltpu.sync_copy(
                zer.at[pl.ds(o, n), :], acc.at[pl.ds(o, n), :]))
        plsc.subcore_barrier()
        j0 = w * CPW
        cnt = jnp.minimum(CPW, NCHUNK - j0)
        for ea, coli, acc in ((ea0, col0, a0), (ea1, col1, a1), (ea2, col2, a2)):

            @pl.when(w < NW - 1)
            def _(coli=coli):
                o = pl.multiple_of(w * CPW, 2)
                pltpu.sync_copy(coli.at[pl.ds(o, CPW), :], colb)

            @pl.when(w == NW - 1)
            def _(coli=coli):
                o = (NW - 1) * CPW
                n = NCHUNK - o
                pltpu.sync_copy(coli.at[pl.ds(o, n), :], colb.at[pl.ds(0, n), :])

            def start(i, buf, sem, ea=ea):
                pltpu.async_copy(ea.at[pl.ds((j0 + i) * CH, CH), :], buf, sem)

            def finish(i, buf, sem, ea=ea, acc=acc):
                pltpu.make_async_copy(ea.at[pl.ds((j0 + i) * CH, CH), :],
                                      buf, sem).wait()
                pltpu.sync_copy(buf, acc.at[colb.at[i]], add=True)

            start(0, ebuf0, sem0)

            def pair(ip, carry, start=start, finish=finish):
                i0 = ip * 2
                i1 = i0 + 1
                i2 = i0 + 2

                @pl.when(i1 < cnt)
                def _():
                    start(i1, ebuf1, sem1)

                @pl.when(i0 < cnt)
                def _():
                    finish(i0, ebuf0, sem0)

                @pl.when(i2 < cnt)
                def _():
                    start(i2, ebuf0, sem0)

                @pl.when(i1 < cnt)
                def _():
                    finish(i1, ebuf1, sem1)

                return carry

            lax.fori_loop(0, (CPW + 1) // 2, pair, 0)
        plsc.subcore_barrier()
        for g, acc in enumerate((a0, a1, a2)):
            _striped(s, lambda o, n, g=g, acc=acc: pltpu.sync_copy(
                acc.at[pl.ds(o, n), :], out.at[c, g, pl.ds(o, n), :]))

    return _gather_add, _edge_seg


def _dot(a, b):
    return jnp.dot(a, b, preferred_element_type=jnp.float32)


def _enc_body(x_ref, w_ref, b_ref, o_ref):
    y = _dot(x_ref[...], w_ref[...]) + b_ref[...]
    o_ref[0] = y[:, :32]
    o_ref[1] = y[:, 32:]


def _encode(xraw, w, b):
    return pl.pallas_call(
        _enc_body,
        grid=(NBLK,),
        in_specs=[pl.BlockSpec((BLK, 17), lambda i: (i, 0)),
                  pl.BlockSpec((17, 64), lambda i: (0, 0)),
                  pl.BlockSpec((1, 64), lambda i: (0, 0))],
        out_specs=pl.BlockSpec((2, BLK, 32), lambda i: (0, i, 0)),
        out_shape=jax.ShapeDtypeStruct((2, N, 32), jnp.float32),
    )(xraw, w, b)


def _k1_body(xst, ax, sacc, wer, wec, m7, v7, w1x, w1e, b1, w2x, w2a, b2,
             x1st, s1o):
    x0 = jnp.concatenate([xst[0], xst[1]], axis=1)
    ax0 = jnp.concatenate([ax[0], ax[1]], axis=1)
    sc = sacc[0, 0] + sacc[1, 0]
    deg = sc[:, 7:8]
    s1 = (_dot(ax0, wer[...]) + _dot(deg * x0, wec[...])
          + _dot(sc[:, 0:7], m7[...]) + deg * v7[...])
    agg1 = _dot(ax0, w1x[...]) + _dot(s1, w1e[...]) + deg * b1[...]
    x1 = _dot(x0, w2x[...]) + _dot(agg1, w2a[...]) + b2[...]
    x1st[0] = x1[:, :32]
    x1st[1] = x1[:, 32:]
    s1o[...] = s1


def _round1(g, xst, ax0, sacc, ws):
    wspec = [pl.BlockSpec(w.shape, lambda i: (0,) * w.ndim) for w in ws]
    return pl.pallas_call(
        _k1_body,
        grid=(NBLK,),
        in_specs=[pl.BlockSpec((2, BLK, 32), lambda i: (0, i, 0)),
                  pl.BlockSpec((2, BLK, 32), lambda i: (0, i, 0)),
                  pl.BlockSpec((2, 1, BLK, 8), lambda i, g=g: (0, g, i, 0))]
        + wspec,
        out_specs=[pl.BlockSpec((2, BLK, 32), lambda i: (0, i, 0)),
                   pl.BlockSpec((BLK, 32), lambda i: (i, 0))],
        out_shape=[jax.ShapeDtypeStruct((2, N, 32), jnp.float32),
                   jax.ShapeDtypeStruct((N, 32), jnp.float32)],
    )(xst, ax0, sacc, *ws)


def _k2_body(x1st, ax, s1, sacc, bt, wer, wec, wea, bev, w1x, w1e, b1,
             w2x, w2a, b2, nacc, eacc):
    i = pl.program_id(0)
    x1 = jnp.concatenate([x1st[0], x1st[1]], axis=1)
    ax1 = jnp.concatenate([ax[0], ax[1]], axis=1)
    sc = sacc[0, 0] + sacc[1, 0]
    deg = sc[:, 7:8]
    s2 = (_dot(ax1, wer[...]) + _dot(deg * x1, wec[...])
          + _dot(s1[...], wea[...]) + deg * bev[...])
    agg2 = _dot(ax1, w1x[...]) + _dot(s2, w1e[...]) + deg * b1[...]
    x2 = _dot(x1, w2x[...]) + _dot(agg2, w2a[...]) + b2[...]
    bvec = bt[0]
    oh = (lax.broadcasted_iota(jnp.int32, (B, 1), 0) == bvec).astype(jnp.float32)
    part_n = _dot(oh, x2)
    part_e = _dot(oh, s2)

    @pl.when(i == 0)
    def _():
        nacc[...] = jnp.zeros_like(nacc)
        eacc[...] = jnp.zeros_like(eacc)

    nacc[...] += part_n
    eacc[...] += part_e


def _round2(g, x1st, ax1, s1, sacc, bt3, ws):
    wspec = [pl.BlockSpec(w.shape, lambda i: (0,) * w.ndim) for w in ws]
    return pl.pallas_call(
        _k2_body,
        grid=(NBLK,),
        in_specs=[pl.BlockSpec((2, BLK, 32), lambda i: (0, i, 0)),
                  pl.BlockSpec((2, BLK, 32), lambda i: (0, i, 0)),
                  pl.BlockSpec((BLK, 32), lambda i: (i, 0)),
                  pl.BlockSpec((2, 1, BLK, 8), lambda i, g=g: (0, g, i, 0)),
                  pl.BlockSpec((1, 1, BLK), lambda i: (i, 0, 0))]
        + wspec,
        out_specs=[pl.BlockSpec((B, 64), lambda i: (0, 0)),
                   pl.BlockSpec((B, 32), lambda i: (0, 0))],
        out_shape=[jax.ShapeDtypeStruct((B, 64), jnp.float32),
                   jax.ShapeDtypeStruct((B, 32), jnp.float32)],
    )(x1st, ax1, s1, sacc, bt3, *ws)


def _fin_body(na0, ea0, na1, ea1, na2, ea2, up1, up2, upm, tt,
              wgn, wge, bg, wn2, bn2, we2t, we2d, be2,
              w3er, w3ec, w3ea, b3e, w31x, w31e, b31, w32x, w32a, b32,
              w3gn, w3ge, b3g, wl1, bl1, lng, lnb, wl2, bl2, out):
    xs = []
    for na, ea in ((na0, ea0), (na1, ea1), (na2, ea2)):
        u = _dot(na[...], wgn[...]) + _dot(ea[...], wge[...]) + bg[...]
        xs.append(_dot(u, wn2[...]) + bn2[...])
    x0, x1, x2 = xs
    p1d = up1[...] / upm[...]
    p2d = up2[...] / upm[...]
    t = tt[...]
    c1 = _dot(t, we2t[...]) + _dot(p1d, we2d[...]) + be2[...]
    c2 = _dot(t, we2t[...]) + _dot(p2d, we2d[...]) + be2[...]
    e0, e1, e2, e3 = c1, c1, c2, c2
    for _ in range(2):
        n0 = _dot(x0, w3er[...]) + _dot(x2, w3ec[...]) + _dot(e0, w3ea[...]) + b3e[...]
        n1 = _dot(x2, w3er[...]) + _dot(x0, w3ec[...]) + _dot(e1, w3ea[...]) + b3e[...]
        n2 = _dot(x1, w3er[...]) + _dot(x2, w3ec[...]) + _dot(e2, w3ea[...]) + b3e[...]
        n3 = _dot(x2, w3er[...]) + _dot(x1, w3ec[...]) + _dot(e3, w3ea[...]) + b3e[...]
        h0 = _dot(x0, w31x[...]) + _dot(n0, w31e[...]) + b31[...]
        h1 = _dot(x2, w31x[...]) + _dot(n1, w31e[...]) + b31[...]
        h2 = _dot(x1, w31x[...]) + _dot(n2, w31e[...]) + b31[...]
        h3 = _dot(x2, w31x[...]) + _dot(n3, w31e[...]) + b31[...]
        x0 = _dot(x0, w32x[...]) + _dot(h1, w32a[...]) + b32[...]
        x1 = _dot(x1, w32x[...]) + _dot(h3, w32a[...]) + b32[...]
        x2 = _dot(x2, w32x[...]) + _dot(h0 + h2, w32a[...]) + b32[...]
        e0, e1, e2, e3 = n0, n1, n2, n3
    gamma = (_dot(x0 + x1 + x2, w3gn[...])
             + _dot(e0 + e1 + e2 + e3, w3ge[...]) + b3g[...])
    h = _dot(gamma, wl1[...]) + bl1[...]
    h = jnp.where(h > 0, h, 0.01 * h)
    mu = jnp.mean(h, axis=-1, keepdims=True)
    d = h - mu
    var = jnp.mean(d * d, axis=-1, keepdims=True)
    h = d / jnp.sqrt(var + 1e-5) * lng[...] + lnb[...]
    out[...] = _dot(h, wl2[...]) + bl2[...]


def _final(args):
    specs = [pl.BlockSpec(a.shape, lambda i: (0,) * a.ndim) for a in args]
    return pl.pallas_call(
        _fin_body,
        grid=(1,),
        in_specs=specs,
        out_specs=pl.BlockSpec((B, 15), lambda i: (0, 0)),
        out_shape=jax.ShapeDtypeStruct((B, 15), jnp.float32),
    )(*args)


def kernel(x_p1, edge_index_p1, edge_attr_p1, u_p1, batch_p1,
           x_p2, edge_index_p2, edge_attr_p2, u_p2, batch_p2,
           x_pm, edge_index_pm, edge_attr_pm, u_pm, batch_pm,
           Temperature, params):
    p = params
    f32 = jnp.float32

    wer = p["m1_edge"]["W"][0:64]
    wec = p["m1_edge"]["W"][64:128]
    wea = p["m1_edge"]["W"][128:160]
    be = p["m1_edge"]["b"]
    w1x = p["m1_n1"]["W"][0:64]
    w1e = p["m1_n1"]["W"][64:96]
    b1 = p["m1_n1"]["b"][None, :]
    w2x = p["m1_n2"]["W"][0:64]
    w2a = p["m1_n2"]["W"][64:128]
    b2 = p["m1_n2"]["b"][None, :]
    m7 = p["enc_e1"]["W"] @ wea
    v7 = (p["enc_e1"]["b"] @ wea + be)[None, :]
    ws1 = [wer, wec, m7, v7, w1x, w1e, b1, w2x, w2a, b2]
    ws2 = [wer, wec, wea, be[None, :], w1x, w1e, b1, w2x, w2a, b2]

    graphs = [(x_p1, edge_index_p1, edge_attr_p1, batch_p1),
              (x_p2, edge_index_p2, edge_attr_p2, batch_p2),
              (x_pm, edge_index_pm, edge_attr_pm, batch_pm)]
    ones_col = jnp.ones((E, 1), f32)
    ea8s = [jnp.concatenate([g[2], ones_col], axis=1) for g in graphs]
    cols = [g[1][1].reshape(NCHUNK, CH) for g in graphs]
    rows = [g[1][0].reshape(NCHUNK, CH) for g in graphs]
    zer8 = jnp.zeros((N, 8), f32)
    zer32 = jnp.zeros((N, 32), f32)

    sc_gather_add, sc_edge_seg = _sc_kernels()
    sacc = sc_edge_seg(ea8s[0], cols[0], ea8s[1], cols[1], ea8s[2], cols[2],
                       zer8)

    wenc = p["enc_n1"]["W"]
    benc = p["enc_n1"]["b"][None, :]
    fin_args = []
    for g in range(3):
        xraw, _, _, bt = graphs[g]
        xst0 = _encode(xraw, wenc, benc)
        ax0 = sc_gather_add(xst0, rows[g], cols[g], zer32)
        x1st, s1 = _round1(g, xst0, ax0, sacc, ws1)
        ax1 = sc_gather_add(x1st, rows[g], cols[g], zer32)
        bt3 = bt.reshape(NBLK, 1, BLK)
        nacc, eacc = _round2(g, x1st, ax1, s1, sacc, bt3, ws2)
        fin_args += [nacc, eacc]

    fin_args += [u_p1[:, None], u_p2[:, None], u_pm[:, None],
                 Temperature[:, None],
                 p["m1_glob"]["W"][0:64], p["m1_glob"]["W"][64:96],
                 p["m1_glob"]["b"][None, :],
                 p["enc_n2"]["W"], p["enc_n2"]["b"][None, :],
                 p["enc_e2"]["W"][0:1], p["enc_e2"]["W"][1:2],
                 p["enc_e2"]["b"][None, :],
                 p["m3_edge"]["W"][0:128], p["m3_edge"]["W"][128:256],
                 p["m3_edge"]["W"][256:288], p["m3_edge"]["b"][None, :],
                 p["m3_n1"]["W"][0:128], p["m3_n1"]["W"][128:160],
                 p["m3_n1"]["b"][None, :],
                 p["m3_n2"]["W"][0:128], p["m3_n2"]["W"][128:256],
                 p["m3_n2"]["b"][None, :],
                 p["m3_glob"]["W"][0:128], p["m3_glob"]["W"][128:160],
                 p["m3_glob"]["b"][None, :],
                 p["last1"]["W"], p["last1"]["b"][None, :],
                 p["ln"]["g"][None, :], p["ln"]["b"][None, :],
                 p["last2"]["W"], p["last2"]["b"][None, :]]
    return _final(fin_args)
